# baseline (speedup 1.0000x reference)
import functools

import jax
import jax.numpy as jnp
from jax import lax
from jax.experimental import pallas as pl
from jax.experimental.pallas import tpu as pltpu
from jax.experimental.pallas import tpu_sc as plsc

_T, _H, _E, _I = 2048, 1024, 8, 2048
_B = 128
_NBP = _T // _B + _E
_PT = _NBP * _B
_IC = 2

_INTERP = False


def _router_body(x_ref, rw_ref, eid_ref, gate_ref):
    x = x_ref[...]
    logits = lax.dot_general(x, rw_ref[...], (((1,), (1,)), ((), ())),
                             preferred_element_type=jnp.float32)
    col = lax.broadcasted_iota(jnp.int32, logits.shape, 1)
    masked = jnp.where(col < _E, logits, -1e30)
    maxv = jnp.max(masked, axis=1)
    eid = jnp.min(jnp.where(masked == maxv[:, None], col, _E), axis=1)
    gate_v = jax.nn.sigmoid(maxv)
    eid_ref[...] = eid.reshape(eid_ref.shape).astype(jnp.int32)
    gate_ref[...] = gate_v.reshape(gate_ref.shape)


def _router(x, rw_pad):
    return pl.pallas_call(
        _router_body,
        out_shape=[
            jax.ShapeDtypeStruct((_T // 128, 128), jnp.int32),
            jax.ShapeDtypeStruct((_T // 128, 128), jnp.float32),
        ],
        interpret=_INTERP,
    )(x, rw_pad)


def _g1_body(bexp_ref, xs_ref, pg_ref, w1_ref, w3_ref, hs_ref):
    g = pg_ref[0, 0, :]
    xg = (xs_ref[...] * g[:, None]).astype(jnp.bfloat16)
    w1b = w1_ref[0].astype(jnp.bfloat16)
    w3b = w3_ref[0].astype(jnp.bfloat16)
    h1 = lax.dot_general(xg, w1b, (((1,), (1,)), ((), ())),
                         preferred_element_type=jnp.float32)
    h3 = lax.dot_general(xg, w3b, (((1,), (1,)), ((), ())),
                         preferred_element_type=jnp.float32)
    hs_ref[...] = (h1 * jax.nn.sigmoid(h1)) * h3


def _g1(bexp, xs, pg2d, w1, w3):
    icw = _I // _IC
    return pl.pallas_call(
        _g1_body,
        grid_spec=pltpu.PrefetchScalarGridSpec(
            num_scalar_prefetch=1,
            grid=(_IC, _NBP),
            in_specs=[
                pl.BlockSpec((_B, _H), lambda ic, b, bexp: (b, 0)),
                pl.BlockSpec((1, 1, _B), lambda ic, b, bexp: (b, 0, 0)),
                pl.BlockSpec((1, icw, _H), lambda ic, b, bexp: (bexp[b], ic, 0)),
                pl.BlockSpec((1, icw, _H), lambda ic, b, bexp: (bexp[b], ic, 0)),
            ],
            out_specs=pl.BlockSpec((_B, icw), lambda ic, b, bexp: (b, ic)),
        ),
        out_shape=jax.ShapeDtypeStruct((_PT, _I), jnp.float32),
        interpret=_INTERP,
    )(bexp, xs, pg2d, w1, w3)


def _g2_body(bexp_ref, hs_ref, w2_ref, ys_ref):
    ys_ref[...] = lax.dot_general(hs_ref[...].astype(jnp.bfloat16),
                                  w2_ref[0].astype(jnp.bfloat16),
                                  (((1,), (1,)), ((), ())),
                                  preferred_element_type=jnp.float32)


def _g2(bexp, hs, w2):
    return pl.pallas_call(
        _g2_body,
        grid_spec=pltpu.PrefetchScalarGridSpec(
            num_scalar_prefetch=1,
            grid=(_NBP,),
            in_specs=[
                pl.BlockSpec((_B, _I), lambda b, bexp: (b, 0)),
                pl.BlockSpec((1, _H, _I), lambda b, bexp: (bexp[b], 0, 0)),
            ],
            out_specs=pl.BlockSpec((_B, _H), lambda b, bexp: (b, 0)),
        ),
        out_shape=jax.ShapeDtypeStruct((_PT, _H), jnp.float32),
        interpret=_INTERP,
    )(bexp, hs, w2)


def _s1_body(x_ref, sw1_ref, sw3_ref, h_ref):
    x = x_ref[...].astype(jnp.bfloat16)
    h1 = lax.dot_general(x, sw1_ref[...].astype(jnp.bfloat16),
                         (((1,), (1,)), ((), ())),
                         preferred_element_type=jnp.float32)
    h3 = lax.dot_general(x, sw3_ref[...].astype(jnp.bfloat16),
                         (((1,), (1,)), ((), ())),
                         preferred_element_type=jnp.float32)
    h_ref[...] = (h1 * jax.nn.sigmoid(h1)) * h3


def _s1(x, sw1, sw3):
    tb = 256
    icw = _I // _IC
    return pl.pallas_call(
        _s1_body,
        grid=(_IC, _T // tb),
        in_specs=[
            pl.BlockSpec((tb, _H), lambda ic, b: (b, 0)),
            pl.BlockSpec((icw, _H), lambda ic, b: (ic, 0)),
            pl.BlockSpec((icw, _H), lambda ic, b: (ic, 0)),
        ],
        out_specs=pl.BlockSpec((tb, icw), lambda ic, b: (b, ic)),
        out_shape=jax.ShapeDtypeStruct((_T, _I), jnp.float32),
        interpret=_INTERP,
    )(x, sw1, sw3)


def _s2_body(h_ref, sw2_ref, yg_ref, y_ref):
    y_ref[...] = yg_ref[...] + lax.dot_general(
        h_ref[...].astype(jnp.bfloat16), sw2_ref[...].astype(jnp.bfloat16),
        (((1,), (1,)), ((), ())),
        preferred_element_type=jnp.float32)


def _s2(hsh, sw2, yg):
    tb = 256
    return pl.pallas_call(
        _s2_body,
        grid=(_T // tb,),
        in_specs=[
            pl.BlockSpec((tb, _I), lambda b: (b, 0)),
            pl.BlockSpec((_H, _I), lambda b: (0, 0)),
            pl.BlockSpec((tb, _H), lambda b: (b, 0)),
        ],
        out_specs=pl.BlockSpec((tb, _H), lambda b: (b, 0)),
        out_shape=jax.ShapeDtypeStruct((_T, _H), jnp.float32),
        interpret=_INTERP,
    )(hsh, sw2, yg)


def _sc_mesh():
    return plsc.VectorSubcoreMesh(core_axis_name="c", subcore_axis_name="s")


_NW = 32
_L = 16


def _take16(vec, idx):
    dn = lax.GatherDimensionNumbers(offset_dims=(), collapsed_slice_dims=(0,),
                                    start_index_map=(0,))
    return lax.gather(vec, idx[:, None], dn, slice_sizes=(1,),
                      mode=lax.GatherScatterMode.PROMISE_IN_BOUNDS)


def _sc_meta_body(eid_hbm, gate_hbm, pos_hbm, perm_hbm, pgate_hbm, bexp_hbm,
                  eid_v, gate_v, pos_v, perm_v, pgate_v, bexp_v, sem):
    wid = lax.axis_index("s") * 2 + lax.axis_index("c")

    @pl.when(wid == 0)
    def _():
        pltpu.sync_copy(eid_hbm, eid_v)
        pltpu.sync_copy(gate_hbm, gate_v)
        lane = lax.broadcasted_iota(jnp.int32, (_L,), 0)

        def count_step(i, c):
            ech = eid_v[pl.ds(i * _L, _L)]
            for e in range(_E):
                cnt = jnp.sum((ech == e).astype(jnp.int32))
                c = c + jnp.where(lane == e, cnt, 0)
            return c

        counts = lax.fori_loop(0, _T // _L, count_step,
                               jnp.zeros((_L,), jnp.int32))
        aligned = ((counts + (_B - 1)) >> 7) << 7
        incl = plsc.cumsum(aligned)
        excl = incl - aligned
        startblk = excl >> 7

        for k in range(2):
            biota = lane + k * _L
            acc = jnp.zeros((_L,), jnp.int32)
            for e in range(_E):
                sb = _take16(startblk, jnp.full((_L,), e, jnp.int32))
                acc = acc + jnp.where(biota >= sb, 1, 0)
            bexp_v[pl.ds(k * _L, _L)] = jnp.maximum(acc - 1, 0)

        def zero_step(j, _):
            perm_v[pl.ds(j * _L, _L)] = jnp.zeros((_L,), jnp.int32)
            pgate_v[pl.ds(j * _L, _L)] = jnp.zeros((_L,), jnp.float32)
            return 0

        lax.fori_loop(0, _PT // _L, zero_step, 0)

        def pos_step(i, c2):
            ech = eid_v[pl.ds(i * _L, _L)]
            gch = gate_v[pl.ds(i * _L, _L)]
            base = _take16(excl + c2, ech)
            within = jnp.zeros((_L,), jnp.int32)
            cadd = jnp.zeros((_L,), jnp.int32)
            for e in range(_E):
                m = ech == e
                mi = m.astype(jnp.int32)
                cs = plsc.cumsum(mi)
                within = within + mi * cs
                cadd = cadd + jnp.where(lane == e, jnp.sum(mi), 0)
            posch = base + within - 1
            pos_v[pl.ds(i * _L, _L)] = posch
            plsc.store_scatter(perm_v, [posch], lane + i * _L)
            plsc.store_scatter(pgate_v, [posch], gch)
            return c2 + cadd

        lax.fori_loop(0, _T // _L, pos_step, jnp.zeros((_L,), jnp.int32))

        pltpu.sync_copy(pos_v, pos_hbm)
        pltpu.sync_copy(perm_v, perm_hbm)
        pltpu.sync_copy(pgate_v, pgate_hbm)
        pltpu.sync_copy(bexp_v, bexp_hbm)


def _sc_meta(eid, gate):
    f = pl.kernel(
        _sc_meta_body,
        out_type=[
            jax.ShapeDtypeStruct((_T,), jnp.int32),
            jax.ShapeDtypeStruct((_PT,), jnp.int32),
            jax.ShapeDtypeStruct((_PT,), jnp.float32),
            jax.ShapeDtypeStruct((32,), jnp.int32),
        ],
        mesh=_sc_mesh(),
        compiler_params=pltpu.CompilerParams(needs_layout_passes=False),
        scratch_types=[
            pltpu.VMEM((_T,), jnp.int32),
            pltpu.VMEM((_T,), jnp.float32),
            pltpu.VMEM((_T,), jnp.int32),
            pltpu.VMEM((_PT,), jnp.int32),
            pltpu.VMEM((_PT,), jnp.float32),
            pltpu.VMEM((32,), jnp.int32),
            pltpu.SemaphoreType.DMA,
        ],
    )
    return f(eid, gate)


_NCH = 4


def _sc_gather_body(src_hbm, idx_hbm, out_hbm, idx_v,
                    b0, b1, b2, b3, g0, g1, g2, g3, w0, w1, w2, w3):
    wid = lax.axis_index("s") * 2 + lax.axis_index("c")
    n = idx_v.shape[0]
    ch = n // _NCH
    base = wid * n
    pltpu.sync_copy(idx_hbm.at[pl.ds(base, n)], idx_v)
    bufs, gsems, wsems = [b0, b1, b2, b3], [g0, g1, g2, g3], [w0, w1, w2, w3]
    gd = [pltpu.async_copy(src_hbm.at[idx_v.at[pl.ds(c * ch, ch)]],
                           bufs[c], gsems[c]) for c in range(_NCH)]
    wd = []
    for c in range(_NCH):
        gd[c].wait()
        wd.append(pltpu.async_copy(bufs[c],
                                   out_hbm.at[pl.ds(base + c * ch, ch)],
                                   wsems[c]))
    for c in range(_NCH):
        wd[c].wait()


def _sc_gather(src, idx, n_out):
    rows_per = n_out // _NW
    ch = rows_per // _NCH
    f = pl.kernel(
        _sc_gather_body,
        out_type=jax.ShapeDtypeStruct((n_out, _H), jnp.float32),
        mesh=_sc_mesh(),
        compiler_params=pltpu.CompilerParams(needs_layout_passes=False),
        scratch_types=(
            [pltpu.VMEM((rows_per,), jnp.int32)]
            + [pltpu.VMEM((ch, _H), jnp.float32) for _ in range(_NCH)]
            + [pltpu.SemaphoreType.DMA for _ in range(2 * _NCH)]
        ),
    )
    return f(src, idx)


def kernel(hidden_states, router_w, w1, w3, w2, sw1, sw3, sw2):
    x = hidden_states
    rw_pad = jnp.zeros((128, _H), jnp.float32).at[:_E].set(router_w)
    eid2d, gate2d = _router(x, rw_pad)
    eid = eid2d.reshape(_T)
    gate = gate2d.reshape(_T)

    pos, perm, pgate, bexp32 = _sc_meta(eid, gate)
    bexp = bexp32[:_NBP]
    pg2d = pgate.reshape(_NBP, 1, _B)

    xs = _sc_gather(x, perm, _PT)
    hs = _g1(bexp, xs, pg2d, w1, w3)
    ys = _g2(bexp, hs, w2)
    yg = _sc_gather(ys, pos, _T)

    hsh = _s1(x, sw1, sw3)
    return _s2(hsh, sw2, yg)


# speedup vs baseline: 1.0049x; 1.0049x over previous
import functools

import jax
import jax.numpy as jnp
from jax import lax
from jax.experimental import pallas as pl
from jax.experimental.pallas import tpu as pltpu
from jax.experimental.pallas import tpu_sc as plsc

_T, _H, _E, _I = 2048, 1024, 8, 2048
_B = 256
_BLOG = 8
_NBP = _T // _B + _E
_PT = _NBP * _B
_IC = 2

_INTERP = False


def _router_body(x_ref, rw_ref, eid_ref, gate_ref):
    x = x_ref[...]
    logits = lax.dot_general(x, rw_ref[...], (((1,), (1,)), ((), ())),
                             preferred_element_type=jnp.float32)
    col = lax.broadcasted_iota(jnp.int32, logits.shape, 1)
    masked = jnp.where(col < _E, logits, -1e30)
    maxv = jnp.max(masked, axis=1)
    eid = jnp.min(jnp.where(masked == maxv[:, None], col, _E), axis=1)
    gate_v = jax.nn.sigmoid(maxv)
    eid_ref[...] = eid.reshape(eid_ref.shape).astype(jnp.int32)
    gate_ref[...] = gate_v.reshape(gate_ref.shape)


def _router(x, rw_pad):
    return pl.pallas_call(
        _router_body,
        out_shape=[
            jax.ShapeDtypeStruct((_T // 128, 128), jnp.int32),
            jax.ShapeDtypeStruct((_T // 128, 128), jnp.float32),
        ],
        interpret=_INTERP,
    )(x, rw_pad)


def _g1_body(bexp_ref, xs_ref, pg_ref, w1_ref, w3_ref, hs_ref):
    g = pg_ref[0, 0, :]
    xg = (xs_ref[...] * g[:, None]).astype(jnp.bfloat16)
    w1b = w1_ref[0].astype(jnp.bfloat16)
    w3b = w3_ref[0].astype(jnp.bfloat16)
    h1 = lax.dot_general(xg, w1b, (((1,), (1,)), ((), ())),
                         preferred_element_type=jnp.float32)
    h3 = lax.dot_general(xg, w3b, (((1,), (1,)), ((), ())),
                         preferred_element_type=jnp.float32)
    hs_ref[...] = (h1 * jax.nn.sigmoid(h1)) * h3


def _g1(bexp, xs, pg2d, w1, w3):
    icw = _I // _IC
    return pl.pallas_call(
        _g1_body,
        grid_spec=pltpu.PrefetchScalarGridSpec(
            num_scalar_prefetch=1,
            grid=(_IC, _NBP),
            in_specs=[
                pl.BlockSpec((_B, _H), lambda ic, b, bexp: (b, 0)),
                pl.BlockSpec((1, 1, _B), lambda ic, b, bexp: (b, 0, 0)),
                pl.BlockSpec((1, icw, _H), lambda ic, b, bexp: (bexp[b], ic, 0)),
                pl.BlockSpec((1, icw, _H), lambda ic, b, bexp: (bexp[b], ic, 0)),
            ],
            out_specs=pl.BlockSpec((_B, icw), lambda ic, b, bexp: (b, ic)),
        ),
        out_shape=jax.ShapeDtypeStruct((_PT, _I), jnp.float32),
        interpret=_INTERP,
    )(bexp, xs, pg2d, w1, w3)


def _g2_body(bexp_ref, hs_ref, w2_ref, ys_ref):
    ys_ref[...] = lax.dot_general(hs_ref[...].astype(jnp.bfloat16),
                                  w2_ref[0].astype(jnp.bfloat16),
                                  (((1,), (1,)), ((), ())),
                                  preferred_element_type=jnp.float32)


def _g2(bexp, hs, w2):
    return pl.pallas_call(
        _g2_body,
        grid_spec=pltpu.PrefetchScalarGridSpec(
            num_scalar_prefetch=1,
            grid=(_NBP,),
            in_specs=[
                pl.BlockSpec((_B, _I), lambda b, bexp: (b, 0)),
                pl.BlockSpec((1, _H, _I), lambda b, bexp: (bexp[b], 0, 0)),
            ],
            out_specs=pl.BlockSpec((_B, _H), lambda b, bexp: (b, 0)),
        ),
        out_shape=jax.ShapeDtypeStruct((_PT, _H), jnp.float32),
        interpret=_INTERP,
    )(bexp, hs, w2)


def _s1_body(x_ref, sw1_ref, sw3_ref, h_ref):
    x = x_ref[...].astype(jnp.bfloat16)
    h1 = lax.dot_general(x, sw1_ref[...].astype(jnp.bfloat16),
                         (((1,), (1,)), ((), ())),
                         preferred_element_type=jnp.float32)
    h3 = lax.dot_general(x, sw3_ref[...].astype(jnp.bfloat16),
                         (((1,), (1,)), ((), ())),
                         preferred_element_type=jnp.float32)
    h_ref[...] = (h1 * jax.nn.sigmoid(h1)) * h3


def _s1(x, sw1, sw3):
    tb = 256
    icw = _I // _IC
    return pl.pallas_call(
        _s1_body,
        grid=(_IC, _T // tb),
        in_specs=[
            pl.BlockSpec((tb, _H), lambda ic, b: (b, 0)),
            pl.BlockSpec((icw, _H), lambda ic, b: (ic, 0)),
            pl.BlockSpec((icw, _H), lambda ic, b: (ic, 0)),
        ],
        out_specs=pl.BlockSpec((tb, icw), lambda ic, b: (b, ic)),
        out_shape=jax.ShapeDtypeStruct((_T, _I), jnp.float32),
        interpret=_INTERP,
    )(x, sw1, sw3)


def _s2_body(h_ref, sw2_ref, yg_ref, y_ref):
    y_ref[...] = yg_ref[...] + lax.dot_general(
        h_ref[...].astype(jnp.bfloat16), sw2_ref[...].astype(jnp.bfloat16),
        (((1,), (1,)), ((), ())),
        preferred_element_type=jnp.float32)


def _s2(hsh, sw2, yg):
    tb = 256
    return pl.pallas_call(
        _s2_body,
        grid=(_T // tb,),
        in_specs=[
            pl.BlockSpec((tb, _I), lambda b: (b, 0)),
            pl.BlockSpec((_H, _I), lambda b: (0, 0)),
            pl.BlockSpec((tb, _H), lambda b: (b, 0)),
        ],
        out_specs=pl.BlockSpec((tb, _H), lambda b: (b, 0)),
        out_shape=jax.ShapeDtypeStruct((_T, _H), jnp.float32),
        interpret=_INTERP,
    )(hsh, sw2, yg)


def _sc_mesh():
    return plsc.VectorSubcoreMesh(core_axis_name="c", subcore_axis_name="s")


_NW = 32
_L = 16


def _take16(vec, idx):
    dn = lax.GatherDimensionNumbers(offset_dims=(), collapsed_slice_dims=(0,),
                                    start_index_map=(0,))
    return lax.gather(vec, idx[:, None], dn, slice_sizes=(1,),
                      mode=lax.GatherScatterMode.PROMISE_IN_BOUNDS)


def _sc_meta_body(eid_hbm, gate_hbm, pos_hbm, perm_hbm, pgate_hbm, bexp_hbm,
                  eid_v, gate_v, pos_v, perm_v, pgate_v, bexp_v, sem):
    wid = lax.axis_index("s") * 2 + lax.axis_index("c")

    @pl.when(wid == 0)
    def _():
        pltpu.sync_copy(eid_hbm, eid_v)
        pltpu.sync_copy(gate_hbm, gate_v)
        lane = lax.broadcasted_iota(jnp.int32, (_L,), 0)

        def count_step(i, c):
            ech = eid_v[pl.ds(i * _L, _L)]
            for e in range(_E):
                cnt = jnp.sum((ech == e).astype(jnp.int32))
                c = c + jnp.where(lane == e, cnt, 0)
            return c

        counts = lax.fori_loop(0, _T // _L, count_step,
                               jnp.zeros((_L,), jnp.int32))
        aligned = ((counts + (_B - 1)) >> _BLOG) << _BLOG
        incl = plsc.cumsum(aligned)
        excl = incl - aligned
        startblk = excl >> _BLOG

        for k in range(2):
            biota = lane + k * _L
            acc = jnp.zeros((_L,), jnp.int32)
            for e in range(_E):
                sb = _take16(startblk, jnp.full((_L,), e, jnp.int32))
                acc = acc + jnp.where(biota >= sb, 1, 0)
            bexp_v[pl.ds(k * _L, _L)] = jnp.maximum(acc - 1, 0)

        def zero_step(j, _):
            perm_v[pl.ds(j * _L, _L)] = jnp.zeros((_L,), jnp.int32)
            pgate_v[pl.ds(j * _L, _L)] = jnp.zeros((_L,), jnp.float32)
            return 0

        lax.fori_loop(0, _PT // _L, zero_step, 0)

        def pos_step(i, c2):
            ech = eid_v[pl.ds(i * _L, _L)]
            gch = gate_v[pl.ds(i * _L, _L)]
            base = _take16(excl + c2, ech)
            within = jnp.zeros((_L,), jnp.int32)
            cadd = jnp.zeros((_L,), jnp.int32)
            for e in range(_E):
                m = ech == e
                mi = m.astype(jnp.int32)
                cs = plsc.cumsum(mi)
                within = within + mi * cs
                cadd = cadd + jnp.where(lane == e, jnp.sum(mi), 0)
            posch = base + within - 1
            pos_v[pl.ds(i * _L, _L)] = posch
            plsc.store_scatter(perm_v, [posch], lane + i * _L)
            plsc.store_scatter(pgate_v, [posch], gch)
            return c2 + cadd

        lax.fori_loop(0, _T // _L, pos_step, jnp.zeros((_L,), jnp.int32))

        pltpu.sync_copy(pos_v, pos_hbm)
        pltpu.sync_copy(perm_v, perm_hbm)
        pltpu.sync_copy(pgate_v, pgate_hbm)
        pltpu.sync_copy(bexp_v, bexp_hbm)


def _sc_meta(eid, gate):
    f = pl.kernel(
        _sc_meta_body,
        out_type=[
            jax.ShapeDtypeStruct((_T,), jnp.int32),
            jax.ShapeDtypeStruct((_PT,), jnp.int32),
            jax.ShapeDtypeStruct((_PT,), jnp.float32),
            jax.ShapeDtypeStruct((32,), jnp.int32),
        ],
        mesh=_sc_mesh(),
        compiler_params=pltpu.CompilerParams(needs_layout_passes=False),
        scratch_types=[
            pltpu.VMEM((_T,), jnp.int32),
            pltpu.VMEM((_T,), jnp.float32),
            pltpu.VMEM((_T,), jnp.int32),
            pltpu.VMEM((_PT,), jnp.int32),
            pltpu.VMEM((_PT,), jnp.float32),
            pltpu.VMEM((32,), jnp.int32),
            pltpu.SemaphoreType.DMA,
        ],
    )
    return f(eid, gate)


_NBUF = 4
_CH = 16


def _sc_gather_body(src_hbm, idx_hbm, out_hbm, idx_v,
                    b0, b1, b2, b3, g0, g1, g2, g3, w0, w1, w2, w3):
    wid = lax.axis_index("s") * 2 + lax.axis_index("c")
    n = idx_v.shape[0]
    nch = n // _CH
    base = wid * n
    pltpu.sync_copy(idx_hbm.at[pl.ds(base, n)], idx_v)
    bufs, gsems, wsems = [b0, b1, b2, b3], [g0, g1, g2, g3], [w0, w1, w2, w3]
    gd, wd = {}, {}
    for c in range(min(_NBUF, nch)):
        gd[c] = pltpu.async_copy(src_hbm.at[idx_v.at[pl.ds(c * _CH, _CH)]],
                                 bufs[c], gsems[c])
    for c in range(nch):
        b = c % _NBUF
        gd[c].wait()
        wd[c] = pltpu.async_copy(bufs[b],
                                 out_hbm.at[pl.ds(base + c * _CH, _CH)],
                                 wsems[b])
        nxt = c + _NBUF
        if nxt < nch:
            wd[c].wait()
            gd[nxt] = pltpu.async_copy(
                src_hbm.at[idx_v.at[pl.ds(nxt * _CH, _CH)]], bufs[b], gsems[b])
    for c in range(max(0, nch - _NBUF), nch):
        wd[c].wait()


def _sc_gather(src, idx, n_out):
    rows_per = n_out // _NW
    f = pl.kernel(
        _sc_gather_body,
        out_type=jax.ShapeDtypeStruct((n_out, _H), jnp.float32),
        mesh=_sc_mesh(),
        compiler_params=pltpu.CompilerParams(needs_layout_passes=False),
        scratch_types=(
            [pltpu.VMEM((rows_per,), jnp.int32)]
            + [pltpu.VMEM((_CH, _H), jnp.float32) for _ in range(_NBUF)]
            + [pltpu.SemaphoreType.DMA for _ in range(2 * _NBUF)]
        ),
    )
    return f(src, idx)


def kernel(hidden_states, router_w, w1, w3, w2, sw1, sw3, sw2):
    x = hidden_states
    rw_pad = jnp.zeros((128, _H), jnp.float32).at[:_E].set(router_w)
    eid2d, gate2d = _router(x, rw_pad)
    eid = eid2d.reshape(_T)
    gate = gate2d.reshape(_T)

    pos, perm, pgate, bexp32 = _sc_meta(eid, gate)
    bexp = bexp32[:_NBP]
    pg2d = pgate.reshape(_NBP, 1, _B)

    xs = _sc_gather(x, perm, _PT)
    hs = _g1(bexp, xs, pg2d, w1, w3)
    ys = _g2(bexp, hs, w2)
    yg = _sc_gather(ys, pos, _T)

    hsh = _s1(x, sw1, sw3)
    return _s2(hsh, sw2, yg)


# speedup vs baseline: 1.0517x; 1.0465x over previous
import functools

import jax
import jax.numpy as jnp
from jax import lax
from jax.experimental import pallas as pl
from jax.experimental.pallas import tpu as pltpu
from jax.experimental.pallas import tpu_sc as plsc

_T, _H, _E, _I = 2048, 1024, 8, 2048
_B = 256
_BLOG = 8
_NBP = _T // _B + _E
_PT = _NBP * _B
_IC = 2

_INTERP = False


def _router_body(x_ref, rw_ref, eid_ref, gate_ref):
    x = x_ref[...]
    logits = lax.dot_general(x, rw_ref[...], (((1,), (1,)), ((), ())),
                             preferred_element_type=jnp.float32)
    col = lax.broadcasted_iota(jnp.int32, logits.shape, 1)
    masked = jnp.where(col < _E, logits, -1e30)
    maxv = jnp.max(masked, axis=1)
    eid = jnp.min(jnp.where(masked == maxv[:, None], col, _E), axis=1)
    gate_v = jax.nn.sigmoid(maxv)
    eid_ref[...] = eid.reshape(eid_ref.shape).astype(jnp.int32)
    gate_ref[...] = gate_v.reshape(gate_ref.shape)


def _router(x, rw_pad):
    return pl.pallas_call(
        _router_body,
        out_shape=[
            jax.ShapeDtypeStruct((_T // 128, 128), jnp.int32),
            jax.ShapeDtypeStruct((_T // 128, 128), jnp.float32),
        ],
        interpret=_INTERP,
    )(x, rw_pad)


def _g1_body(bexp_ref, xs_ref, pg_ref, w1_ref, w3_ref, hs_ref):
    g = pg_ref[0, 0, :]
    xg = (xs_ref[...] * g[:, None]).astype(jnp.bfloat16)
    w1b = w1_ref[0].astype(jnp.bfloat16)
    w3b = w3_ref[0].astype(jnp.bfloat16)
    h1 = lax.dot_general(xg, w1b, (((1,), (1,)), ((), ())),
                         preferred_element_type=jnp.float32)
    h3 = lax.dot_general(xg, w3b, (((1,), (1,)), ((), ())),
                         preferred_element_type=jnp.float32)
    hs_ref[...] = (h1 * jax.nn.sigmoid(h1)) * h3


def _g1(bexp, xs, pg2d, w1, w3):
    icw = _I // _IC
    return pl.pallas_call(
        _g1_body,
        grid_spec=pltpu.PrefetchScalarGridSpec(
            num_scalar_prefetch=1,
            grid=(_IC, _NBP),
            in_specs=[
                pl.BlockSpec((_B, _H), lambda ic, b, bexp: (b, 0)),
                pl.BlockSpec((1, 1, _B), lambda ic, b, bexp: (b, 0, 0)),
                pl.BlockSpec((1, icw, _H), lambda ic, b, bexp: (bexp[b], ic, 0)),
                pl.BlockSpec((1, icw, _H), lambda ic, b, bexp: (bexp[b], ic, 0)),
            ],
            out_specs=pl.BlockSpec((_B, icw), lambda ic, b, bexp: (b, ic)),
        ),
        out_shape=jax.ShapeDtypeStruct((_PT, _I), jnp.float32),
        interpret=_INTERP,
    )(bexp, xs, pg2d, w1, w3)


def _g2_body(bexp_ref, hs_ref, w2_ref, ys_ref):
    ys_ref[...] = lax.dot_general(hs_ref[...].astype(jnp.bfloat16),
                                  w2_ref[0].astype(jnp.bfloat16),
                                  (((1,), (1,)), ((), ())),
                                  preferred_element_type=jnp.float32)


def _g2(bexp, hs, w2):
    return pl.pallas_call(
        _g2_body,
        grid_spec=pltpu.PrefetchScalarGridSpec(
            num_scalar_prefetch=1,
            grid=(_NBP,),
            in_specs=[
                pl.BlockSpec((_B, _I), lambda b, bexp: (b, 0)),
                pl.BlockSpec((1, _H, _I), lambda b, bexp: (bexp[b], 0, 0)),
            ],
            out_specs=pl.BlockSpec((_B, _H), lambda b, bexp: (b, 0)),
        ),
        out_shape=jax.ShapeDtypeStruct((_PT, _H), jnp.float32),
        interpret=_INTERP,
    )(bexp, hs, w2)


def _s1_body(x_ref, sw1_ref, sw3_ref, h_ref):
    x = x_ref[...].astype(jnp.bfloat16)
    h1 = lax.dot_general(x, sw1_ref[...].astype(jnp.bfloat16),
                         (((1,), (1,)), ((), ())),
                         preferred_element_type=jnp.float32)
    h3 = lax.dot_general(x, sw3_ref[...].astype(jnp.bfloat16),
                         (((1,), (1,)), ((), ())),
                         preferred_element_type=jnp.float32)
    h_ref[...] = (h1 * jax.nn.sigmoid(h1)) * h3


def _s1(x, sw1, sw3):
    tb = 256
    icw = _I // _IC
    return pl.pallas_call(
        _s1_body,
        grid=(_IC, _T // tb),
        in_specs=[
            pl.BlockSpec((tb, _H), lambda ic, b: (b, 0)),
            pl.BlockSpec((icw, _H), lambda ic, b: (ic, 0)),
            pl.BlockSpec((icw, _H), lambda ic, b: (ic, 0)),
        ],
        out_specs=pl.BlockSpec((tb, icw), lambda ic, b: (b, ic)),
        out_shape=jax.ShapeDtypeStruct((_T, _I), jnp.float32),
        interpret=_INTERP,
    )(x, sw1, sw3)


def _s2_body(h_ref, sw2_ref, y_ref):
    y_ref[...] = lax.dot_general(
        h_ref[...].astype(jnp.bfloat16), sw2_ref[...].astype(jnp.bfloat16),
        (((1,), (1,)), ((), ())),
        preferred_element_type=jnp.float32)


def _s2(hsh, sw2):
    tb = 256
    return pl.pallas_call(
        _s2_body,
        grid=(_T // tb,),
        in_specs=[
            pl.BlockSpec((tb, _I), lambda b: (b, 0)),
            pl.BlockSpec((_H, _I), lambda b: (0, 0)),
        ],
        out_specs=pl.BlockSpec((tb, _H), lambda b: (b, 0)),
        out_shape=jax.ShapeDtypeStruct((_T, _H), jnp.float32),
        interpret=_INTERP,
    )(hsh, sw2)


def _sc_mesh():
    return plsc.VectorSubcoreMesh(core_axis_name="c", subcore_axis_name="s")


_NW = 32
_L = 16


def _take16(vec, idx):
    dn = lax.GatherDimensionNumbers(offset_dims=(), collapsed_slice_dims=(0,),
                                    start_index_map=(0,))
    return lax.gather(vec, idx[:, None], dn, slice_sizes=(1,),
                      mode=lax.GatherScatterMode.PROMISE_IN_BOUNDS)


def _sc_meta_body(eid_hbm, gate_hbm, pos_hbm, perm_hbm, pgate_hbm, bexp_hbm,
                  eid_v, gate_v, pos_v, perm_v, pgate_v, bexp_v, sem):
    wid = lax.axis_index("s") * 2 + lax.axis_index("c")

    @pl.when(wid == 0)
    def _():
        pltpu.sync_copy(eid_hbm, eid_v)
        pltpu.sync_copy(gate_hbm, gate_v)
        lane = lax.broadcasted_iota(jnp.int32, (_L,), 0)

        def count_step(i, c):
            ech = eid_v[pl.ds(i * _L, _L)]
            for e in range(_E):
                cnt = jnp.sum((ech == e).astype(jnp.int32))
                c = c + jnp.where(lane == e, cnt, 0)
            return c

        counts = lax.fori_loop(0, _T // _L, count_step,
                               jnp.zeros((_L,), jnp.int32))
        aligned = ((counts + (_B - 1)) >> _BLOG) << _BLOG
        incl = plsc.cumsum(aligned)
        excl = incl - aligned
        startblk = excl >> _BLOG

        for k in range(2):
            biota = lane + k * _L
            acc = jnp.zeros((_L,), jnp.int32)
            for e in range(_E):
                sb = _take16(startblk, jnp.full((_L,), e, jnp.int32))
                acc = acc + jnp.where(biota >= sb, 1, 0)
            bexp_v[pl.ds(k * _L, _L)] = jnp.maximum(acc - 1, 0)

        def zero_step(j, _):
            perm_v[pl.ds(j * _L, _L)] = jnp.zeros((_L,), jnp.int32)
            pgate_v[pl.ds(j * _L, _L)] = jnp.zeros((_L,), jnp.float32)
            return 0

        lax.fori_loop(0, _PT // _L, zero_step, 0)

        def pos_step(i, c2):
            ech = eid_v[pl.ds(i * _L, _L)]
            gch = gate_v[pl.ds(i * _L, _L)]
            base = _take16(excl + c2, ech)
            within = jnp.zeros((_L,), jnp.int32)
            cadd = jnp.zeros((_L,), jnp.int32)
            for e in range(_E):
                m = ech == e
                mi = m.astype(jnp.int32)
                cs = plsc.cumsum(mi)
                within = within + mi * cs
                cadd = cadd + jnp.where(lane == e, jnp.sum(mi), 0)
            posch = base + within - 1
            pos_v[pl.ds(i * _L, _L)] = posch
            plsc.store_scatter(perm_v, [posch], lane + i * _L)
            plsc.store_scatter(pgate_v, [posch], gch)
            return c2 + cadd

        lax.fori_loop(0, _T // _L, pos_step, jnp.zeros((_L,), jnp.int32))

        pltpu.sync_copy(pos_v, pos_hbm)
        pltpu.sync_copy(perm_v, perm_hbm)
        pltpu.sync_copy(pgate_v, pgate_hbm)
        pltpu.sync_copy(bexp_v, bexp_hbm)


def _sc_meta(eid, gate):
    f = pl.kernel(
        _sc_meta_body,
        out_type=[
            jax.ShapeDtypeStruct((_T,), jnp.int32),
            jax.ShapeDtypeStruct((_PT,), jnp.int32),
            jax.ShapeDtypeStruct((_PT,), jnp.float32),
            jax.ShapeDtypeStruct((32,), jnp.int32),
        ],
        mesh=_sc_mesh(),
        compiler_params=pltpu.CompilerParams(needs_layout_passes=False),
        scratch_types=[
            pltpu.VMEM((_T,), jnp.int32),
            pltpu.VMEM((_T,), jnp.float32),
            pltpu.VMEM((_T,), jnp.int32),
            pltpu.VMEM((_PT,), jnp.int32),
            pltpu.VMEM((_PT,), jnp.float32),
            pltpu.VMEM((32,), jnp.int32),
            pltpu.SemaphoreType.DMA,
        ],
    )
    return f(eid, gate)


_NBUF = 4
_CH = 16


def _sc_gather_body(src_hbm, idx_hbm, out_hbm, idx_v,
                    b0, b1, b2, b3, g0, g1, g2, g3, w0, w1, w2, w3):
    wid = lax.axis_index("s") * 2 + lax.axis_index("c")
    n = idx_v.shape[0]
    nch = n // _CH
    base = wid * n
    pltpu.sync_copy(idx_hbm.at[pl.ds(base, n)], idx_v)
    bufs, gsems, wsems = [b0, b1, b2, b3], [g0, g1, g2, g3], [w0, w1, w2, w3]
    gd, wd = {}, {}
    for c in range(min(_NBUF, nch)):
        gd[c] = pltpu.async_copy(src_hbm.at[idx_v.at[pl.ds(c * _CH, _CH)]],
                                 bufs[c], gsems[c])
    for c in range(nch):
        b = c % _NBUF
        gd[c].wait()
        wd[c] = pltpu.async_copy(bufs[b],
                                 out_hbm.at[pl.ds(base + c * _CH, _CH)],
                                 wsems[b])
        nxt = c + _NBUF
        if nxt < nch:
            wd[c].wait()
            gd[nxt] = pltpu.async_copy(
                src_hbm.at[idx_v.at[pl.ds(nxt * _CH, _CH)]], bufs[b], gsems[b])
    for c in range(max(0, nch - _NBUF), nch):
        wd[c].wait()


_CHC = 8


def _sc_combine_body(src_hbm, idx_hbm, add_hbm, out_hbm, idx_v,
                     b0, b1, b2, b3, a0, a1, a2, a3,
                     g0, g1, g2, g3, h0, h1, h2, h3, w0, w1, w2, w3):
    wid = lax.axis_index("s") * 2 + lax.axis_index("c")
    n = idx_v.shape[0]
    nch = n // _CHC
    base = wid * n
    pltpu.sync_copy(idx_hbm.at[pl.ds(base, n)], idx_v)
    bufs, abufs = [b0, b1, b2, b3], [a0, a1, a2, a3]
    gsems, asems, wsems = [g0, g1, g2, g3], [h0, h1, h2, h3], [w0, w1, w2, w3]
    gd, ad, wd = {}, {}, {}
    for c in range(min(_NBUF, nch)):
        gd[c] = pltpu.async_copy(src_hbm.at[idx_v.at[pl.ds(c * _CHC, _CHC)]],
                                 bufs[c], gsems[c])
        ad[c] = pltpu.async_copy(add_hbm.at[pl.ds(base + c * _CHC, _CHC)],
                                 abufs[c], asems[c])
    nv = _H // _L
    for c in range(nch):
        b = c % _NBUF
        gd[c].wait()
        ad[c].wait()
        buf, abuf = bufs[b], abufs[b]

        def add_step(r, _, buf=buf, abuf=abuf):
            for k in range(nv):
                sl = pl.ds(k * _L, _L)
                buf[r, sl] = buf[r, sl] + abuf[r, sl]
            return 0

        lax.fori_loop(0, _CHC, add_step, 0)
        wd[c] = pltpu.async_copy(bufs[b],
                                 out_hbm.at[pl.ds(base + c * _CHC, _CHC)],
                                 wsems[b])
        nxt = c + _NBUF
        if nxt < nch:
            wd[c].wait()
            gd[nxt] = pltpu.async_copy(
                src_hbm.at[idx_v.at[pl.ds(nxt * _CHC, _CHC)]], bufs[b], gsems[b])
            ad[nxt] = pltpu.async_copy(
                add_hbm.at[pl.ds(base + nxt * _CHC, _CHC)], abufs[b], asems[b])
    for c in range(max(0, nch - _NBUF), nch):
        wd[c].wait()


def _sc_combine(src, idx, add):
    rows_per = _T // _NW
    f = pl.kernel(
        _sc_combine_body,
        out_type=jax.ShapeDtypeStruct((_T, _H), jnp.float32),
        mesh=_sc_mesh(),
        compiler_params=pltpu.CompilerParams(needs_layout_passes=False),
        scratch_types=(
            [pltpu.VMEM((rows_per,), jnp.int32)]
            + [pltpu.VMEM((_CHC, _H), jnp.float32) for _ in range(2 * _NBUF)]
            + [pltpu.SemaphoreType.DMA for _ in range(3 * _NBUF)]
        ),
    )
    return f(src, idx, add)


def _sc_gather(src, idx, n_out):
    rows_per = n_out // _NW
    f = pl.kernel(
        _sc_gather_body,
        out_type=jax.ShapeDtypeStruct((n_out, _H), jnp.float32),
        mesh=_sc_mesh(),
        compiler_params=pltpu.CompilerParams(needs_layout_passes=False),
        scratch_types=(
            [pltpu.VMEM((rows_per,), jnp.int32)]
            + [pltpu.VMEM((_CH, _H), jnp.float32) for _ in range(_NBUF)]
            + [pltpu.SemaphoreType.DMA for _ in range(2 * _NBUF)]
        ),
    )
    return f(src, idx)


def kernel(hidden_states, router_w, w1, w3, w2, sw1, sw3, sw2):
    x = hidden_states
    eid2d, gate2d = _router(x, router_w)
    eid = eid2d.reshape(_T)
    gate = gate2d.reshape(_T)

    pos, perm, pgate, bexp = _sc_meta(eid, gate)
    pg2d = pgate.reshape(_NBP, 1, _B)

    hsh = _s1(x, sw1, sw3)
    ysh = _s2(hsh, sw2)

    xs = _sc_gather(x, perm, _PT)
    hs = _g1(bexp, xs, pg2d, w1, w3)
    ys = _g2(bexp, hs, w2)

    return _sc_combine(ys, pos, ysh)


# speedup vs baseline: 1.1594x; 1.1025x over previous
import functools

import jax
import jax.numpy as jnp
from jax import lax
from jax.experimental import pallas as pl
from jax.experimental.pallas import tpu as pltpu
from jax.experimental.pallas import tpu_sc as plsc

_T, _H, _E, _I = 2048, 1024, 8, 2048
_B = 256
_BLOG = 8
_NBP = _T // _B + _E
_PT = _NBP * _B
_IC = 2

_INTERP = False


def _router_body(x_ref, rw_ref, eid_ref, gate_ref):
    x = x_ref[...]
    logits = lax.dot_general(x, rw_ref[...], (((1,), (1,)), ((), ())),
                             preferred_element_type=jnp.float32)
    col = lax.broadcasted_iota(jnp.int32, logits.shape, 1)
    masked = jnp.where(col < _E, logits, -1e30)
    maxv = jnp.max(masked, axis=1)
    eid = jnp.min(jnp.where(masked == maxv[:, None], col, _E), axis=1)
    gate_v = jax.nn.sigmoid(maxv)
    eid_ref[...] = eid.reshape(eid_ref.shape).astype(jnp.int32)
    gate_ref[...] = gate_v.reshape(gate_ref.shape)


def _router(x, rw_pad):
    return pl.pallas_call(
        _router_body,
        out_shape=[
            jax.ShapeDtypeStruct((_T // 128, 128), jnp.int32),
            jax.ShapeDtypeStruct((_T // 128, 128), jnp.float32),
        ],
        interpret=_INTERP,
    )(x, rw_pad)


def _g_body(bexp_ref, xs_ref, pg_ref, w1_ref, w3_ref, w2_ref, ys_ref):
    g = pg_ref[0, 0, :]
    xg = (xs_ref[...] * g[:, None]).astype(jnp.bfloat16)
    w1b = w1_ref[0].astype(jnp.bfloat16)
    w3b = w3_ref[0].astype(jnp.bfloat16)
    h1 = lax.dot_general(xg, w1b, (((1,), (1,)), ((), ())),
                         preferred_element_type=jnp.float32)
    h3 = lax.dot_general(xg, w3b, (((1,), (1,)), ((), ())),
                         preferred_element_type=jnp.float32)
    h = ((h1 * jax.nn.sigmoid(h1)) * h3).astype(jnp.bfloat16)
    ys_ref[...] = lax.dot_general(h, w2_ref[0].astype(jnp.bfloat16),
                                  (((1,), (1,)), ((), ())),
                                  preferred_element_type=jnp.float32)


def _g(bexp, xs, pg2d, w1, w3, w2):
    return pl.pallas_call(
        _g_body,
        grid_spec=pltpu.PrefetchScalarGridSpec(
            num_scalar_prefetch=1,
            grid=(_NBP,),
            in_specs=[
                pl.BlockSpec((_B, _H), lambda b, bexp: (b, 0)),
                pl.BlockSpec((1, 1, _B), lambda b, bexp: (b, 0, 0)),
                pl.BlockSpec((1, _I, _H), lambda b, bexp: (bexp[b], 0, 0)),
                pl.BlockSpec((1, _I, _H), lambda b, bexp: (bexp[b], 0, 0)),
                pl.BlockSpec((1, _H, _I), lambda b, bexp: (bexp[b], 0, 0)),
            ],
            out_specs=pl.BlockSpec((_B, _H), lambda b, bexp: (b, 0)),
        ),
        out_shape=jax.ShapeDtypeStruct((_PT, _H), jnp.float32),
        interpret=_INTERP,
    )(bexp, xs, pg2d, w1, w3, w2)


def _g1_body_unused(bexp_ref, xs_ref, pg_ref, w1_ref, w3_ref, hs_ref):
    pass


def _g1(bexp, xs, pg2d, w1, w3):
    icw = _I // _IC
    return pl.pallas_call(
        _g1_body,
        grid_spec=pltpu.PrefetchScalarGridSpec(
            num_scalar_prefetch=1,
            grid=(_IC, _NBP),
            in_specs=[
                pl.BlockSpec((_B, _H), lambda ic, b, bexp: (b, 0)),
                pl.BlockSpec((1, 1, _B), lambda ic, b, bexp: (b, 0, 0)),
                pl.BlockSpec((1, icw, _H), lambda ic, b, bexp: (bexp[b], ic, 0)),
                pl.BlockSpec((1, icw, _H), lambda ic, b, bexp: (bexp[b], ic, 0)),
            ],
            out_specs=pl.BlockSpec((_B, icw), lambda ic, b, bexp: (b, ic)),
        ),
        out_shape=jax.ShapeDtypeStruct((_PT, _I), jnp.float32),
        interpret=_INTERP,
    )(bexp, xs, pg2d, w1, w3)


def _g2_body(bexp_ref, hs_ref, w2_ref, ys_ref):
    ys_ref[...] = lax.dot_general(hs_ref[...].astype(jnp.bfloat16),
                                  w2_ref[0].astype(jnp.bfloat16),
                                  (((1,), (1,)), ((), ())),
                                  preferred_element_type=jnp.float32)


def _g2(bexp, hs, w2):
    return pl.pallas_call(
        _g2_body,
        grid_spec=pltpu.PrefetchScalarGridSpec(
            num_scalar_prefetch=1,
            grid=(_NBP,),
            in_specs=[
                pl.BlockSpec((_B, _I), lambda b, bexp: (b, 0)),
                pl.BlockSpec((1, _H, _I), lambda b, bexp: (bexp[b], 0, 0)),
            ],
            out_specs=pl.BlockSpec((_B, _H), lambda b, bexp: (b, 0)),
        ),
        out_shape=jax.ShapeDtypeStruct((_PT, _H), jnp.float32),
        interpret=_INTERP,
    )(bexp, hs, w2)


def _s1_body(x_ref, sw1_ref, sw3_ref, h_ref):
    x = x_ref[...].astype(jnp.bfloat16)
    h1 = lax.dot_general(x, sw1_ref[...].astype(jnp.bfloat16),
                         (((1,), (1,)), ((), ())),
                         preferred_element_type=jnp.float32)
    h3 = lax.dot_general(x, sw3_ref[...].astype(jnp.bfloat16),
                         (((1,), (1,)), ((), ())),
                         preferred_element_type=jnp.float32)
    h_ref[...] = (h1 * jax.nn.sigmoid(h1)) * h3


def _s1(x, sw1, sw3):
    tb = 256
    icw = _I // _IC
    return pl.pallas_call(
        _s1_body,
        grid=(_IC, _T // tb),
        in_specs=[
            pl.BlockSpec((tb, _H), lambda ic, b: (b, 0)),
            pl.BlockSpec((icw, _H), lambda ic, b: (ic, 0)),
            pl.BlockSpec((icw, _H), lambda ic, b: (ic, 0)),
        ],
        out_specs=pl.BlockSpec((tb, icw), lambda ic, b: (b, ic)),
        out_shape=jax.ShapeDtypeStruct((_T, _I), jnp.float32),
        interpret=_INTERP,
    )(x, sw1, sw3)


def _s2_body(h_ref, sw2_ref, y_ref):
    y_ref[...] = lax.dot_general(
        h_ref[...].astype(jnp.bfloat16), sw2_ref[...].astype(jnp.bfloat16),
        (((1,), (1,)), ((), ())),
        preferred_element_type=jnp.float32)


def _s2(hsh, sw2):
    tb = 256
    return pl.pallas_call(
        _s2_body,
        grid=(_T // tb,),
        in_specs=[
            pl.BlockSpec((tb, _I), lambda b: (b, 0)),
            pl.BlockSpec((_H, _I), lambda b: (0, 0)),
        ],
        out_specs=pl.BlockSpec((tb, _H), lambda b: (b, 0)),
        out_shape=jax.ShapeDtypeStruct((_T, _H), jnp.float32),
        interpret=_INTERP,
    )(hsh, sw2)


def _sc_mesh():
    return plsc.VectorSubcoreMesh(core_axis_name="c", subcore_axis_name="s")


_NW = 32
_L = 16


def _take16(vec, idx):
    dn = lax.GatherDimensionNumbers(offset_dims=(), collapsed_slice_dims=(0,),
                                    start_index_map=(0,))
    return lax.gather(vec, idx[:, None], dn, slice_sizes=(1,),
                      mode=lax.GatherScatterMode.PROMISE_IN_BOUNDS)


def _sc_meta_body(eid_hbm, gate_hbm, pos_hbm, perm_hbm, pgate_hbm, bexp_hbm,
                  eid_v, gate_v, pos_v, perm_v, pgate_v, bexp_v, sem):
    wid = lax.axis_index("s") * 2 + lax.axis_index("c")

    @pl.when(wid == 0)
    def _():
        pltpu.sync_copy(eid_hbm, eid_v)
        pltpu.sync_copy(gate_hbm, gate_v)
        lane = lax.broadcasted_iota(jnp.int32, (_L,), 0)

        def count_step(i, c):
            ech = eid_v[pl.ds(i * _L, _L)]
            for e in range(_E):
                cnt = jnp.sum((ech == e).astype(jnp.int32))
                c = c + jnp.where(lane == e, cnt, 0)
            return c

        counts = lax.fori_loop(0, _T // _L, count_step,
                               jnp.zeros((_L,), jnp.int32))
        aligned = ((counts + (_B - 1)) >> _BLOG) << _BLOG
        incl = plsc.cumsum(aligned)
        excl = incl - aligned
        startblk = excl >> _BLOG

        for k in range(2):
            biota = lane + k * _L
            acc = jnp.zeros((_L,), jnp.int32)
            for e in range(_E):
                sb = _take16(startblk, jnp.full((_L,), e, jnp.int32))
                acc = acc + jnp.where(biota >= sb, 1, 0)
            bexp_v[pl.ds(k * _L, _L)] = jnp.maximum(acc - 1, 0)

        def zero_step(j, _):
            perm_v[pl.ds(j * _L, _L)] = jnp.zeros((_L,), jnp.int32)
            pgate_v[pl.ds(j * _L, _L)] = jnp.zeros((_L,), jnp.float32)
            return 0

        lax.fori_loop(0, _PT // _L, zero_step, 0)

        def pos_step(i, c2):
            ech = eid_v[pl.ds(i * _L, _L)]
            gch = gate_v[pl.ds(i * _L, _L)]
            base = _take16(excl + c2, ech)
            within = jnp.zeros((_L,), jnp.int32)
            cadd = jnp.zeros((_L,), jnp.int32)
            for e in range(_E):
                m = ech == e
                mi = m.astype(jnp.int32)
                cs = plsc.cumsum(mi)
                within = within + mi * cs
                cadd = cadd + jnp.where(lane == e, jnp.sum(mi), 0)
            posch = base + within - 1
            pos_v[pl.ds(i * _L, _L)] = posch
            plsc.store_scatter(perm_v, [posch], lane + i * _L)
            plsc.store_scatter(pgate_v, [posch], gch)
            return c2 + cadd

        lax.fori_loop(0, _T // _L, pos_step, jnp.zeros((_L,), jnp.int32))

        pltpu.sync_copy(pos_v, pos_hbm)
        pltpu.sync_copy(perm_v, perm_hbm)
        pltpu.sync_copy(pgate_v, pgate_hbm)
        pltpu.sync_copy(bexp_v, bexp_hbm)


def _sc_meta(eid, gate):
    f = pl.kernel(
        _sc_meta_body,
        out_type=[
            jax.ShapeDtypeStruct((_T,), jnp.int32),
            jax.ShapeDtypeStruct((_PT,), jnp.int32),
            jax.ShapeDtypeStruct((_PT,), jnp.float32),
            jax.ShapeDtypeStruct((32,), jnp.int32),
        ],
        mesh=_sc_mesh(),
        compiler_params=pltpu.CompilerParams(needs_layout_passes=False),
        scratch_types=[
            pltpu.VMEM((_T,), jnp.int32),
            pltpu.VMEM((_T,), jnp.float32),
            pltpu.VMEM((_T,), jnp.int32),
            pltpu.VMEM((_PT,), jnp.int32),
            pltpu.VMEM((_PT,), jnp.float32),
            pltpu.VMEM((32,), jnp.int32),
            pltpu.SemaphoreType.DMA,
        ],
    )
    return f(eid, gate)


_NBUF = 4
_CH = 16


def _sc_gather_body(src_hbm, idx_hbm, out_hbm, idx_v,
                    b0, b1, b2, b3, g0, g1, g2, g3, w0, w1, w2, w3):
    wid = lax.axis_index("s") * 2 + lax.axis_index("c")
    n = idx_v.shape[0]
    nch = n // _CH
    base = wid * n
    pltpu.sync_copy(idx_hbm.at[pl.ds(base, n)], idx_v)
    bufs, gsems, wsems = [b0, b1, b2, b3], [g0, g1, g2, g3], [w0, w1, w2, w3]
    gd, wd = {}, {}
    for c in range(min(_NBUF, nch)):
        gd[c] = pltpu.async_copy(src_hbm.at[idx_v.at[pl.ds(c * _CH, _CH)]],
                                 bufs[c], gsems[c])
    for c in range(nch):
        b = c % _NBUF
        gd[c].wait()
        wd[c] = pltpu.async_copy(bufs[b],
                                 out_hbm.at[pl.ds(base + c * _CH, _CH)],
                                 wsems[b])
        nxt = c + _NBUF
        if nxt < nch:
            wd[c].wait()
            gd[nxt] = pltpu.async_copy(
                src_hbm.at[idx_v.at[pl.ds(nxt * _CH, _CH)]], bufs[b], gsems[b])
    for c in range(max(0, nch - _NBUF), nch):
        wd[c].wait()


_CHC = 8


def _sc_combine_body(src_hbm, idx_hbm, add_hbm, out_hbm, idx_v,
                     b0, b1, b2, b3, a0, a1, a2, a3,
                     g0, g1, g2, g3, h0, h1, h2, h3, w0, w1, w2, w3):
    wid = lax.axis_index("s") * 2 + lax.axis_index("c")
    n = idx_v.shape[0]
    nch = n // _CHC
    base = wid * n
    pltpu.sync_copy(idx_hbm.at[pl.ds(base, n)], idx_v)
    bufs, abufs = [b0, b1, b2, b3], [a0, a1, a2, a3]
    gsems, asems, wsems = [g0, g1, g2, g3], [h0, h1, h2, h3], [w0, w1, w2, w3]
    gd, ad, wd = {}, {}, {}
    for c in range(min(_NBUF, nch)):
        gd[c] = pltpu.async_copy(src_hbm.at[idx_v.at[pl.ds(c * _CHC, _CHC)]],
                                 bufs[c], gsems[c])
        ad[c] = pltpu.async_copy(add_hbm.at[pl.ds(base + c * _CHC, _CHC)],
                                 abufs[c], asems[c])
    nv = _H // _L
    for c in range(nch):
        b = c % _NBUF
        gd[c].wait()
        ad[c].wait()
        buf, abuf = bufs[b], abufs[b]

        def add_step(r, _, buf=buf, abuf=abuf):
            for k in range(nv):
                sl = pl.ds(k * _L, _L)
                buf[r, sl] = buf[r, sl] + abuf[r, sl]
            return 0

        lax.fori_loop(0, _CHC, add_step, 0)
        wd[c] = pltpu.async_copy(bufs[b],
                                 out_hbm.at[pl.ds(base + c * _CHC, _CHC)],
                                 wsems[b])
        nxt = c + _NBUF
        if nxt < nch:
            wd[c].wait()
            gd[nxt] = pltpu.async_copy(
                src_hbm.at[idx_v.at[pl.ds(nxt * _CHC, _CHC)]], bufs[b], gsems[b])
            ad[nxt] = pltpu.async_copy(
                add_hbm.at[pl.ds(base + nxt * _CHC, _CHC)], abufs[b], asems[b])
    for c in range(max(0, nch - _NBUF), nch):
        wd[c].wait()


def _sc_combine(src, idx, add):
    rows_per = _T // _NW
    f = pl.kernel(
        _sc_combine_body,
        out_type=jax.ShapeDtypeStruct((_T, _H), jnp.float32),
        mesh=_sc_mesh(),
        compiler_params=pltpu.CompilerParams(needs_layout_passes=False),
        scratch_types=(
            [pltpu.VMEM((rows_per,), jnp.int32)]
            + [pltpu.VMEM((_CHC, _H), jnp.float32) for _ in range(2 * _NBUF)]
            + [pltpu.SemaphoreType.DMA for _ in range(3 * _NBUF)]
        ),
    )
    return f(src, idx, add)


def _sc_gather(src, idx, n_out):
    rows_per = n_out // _NW
    f = pl.kernel(
        _sc_gather_body,
        out_type=jax.ShapeDtypeStruct((n_out, _H), jnp.float32),
        mesh=_sc_mesh(),
        compiler_params=pltpu.CompilerParams(needs_layout_passes=False),
        scratch_types=(
            [pltpu.VMEM((rows_per,), jnp.int32)]
            + [pltpu.VMEM((_CH, _H), jnp.float32) for _ in range(_NBUF)]
            + [pltpu.SemaphoreType.DMA for _ in range(2 * _NBUF)]
        ),
    )
    return f(src, idx)


def kernel(hidden_states, router_w, w1, w3, w2, sw1, sw3, sw2):
    x = hidden_states
    eid2d, gate2d = _router(x, router_w)
    eid = eid2d.reshape(_T)
    gate = gate2d.reshape(_T)

    pos, perm, pgate, bexp = _sc_meta(eid, gate)
    pg2d = pgate.reshape(_NBP, 1, _B)

    hsh = _s1(x, sw1, sw3)
    ysh = _s2(hsh, sw2)

    xs = _sc_gather(x, perm, _PT)
    ys = _g(bexp, xs, pg2d, w1, w3, w2)

    return _sc_combine(ys, pos, ysh)


# speedup vs baseline: 1.2281x; 1.0593x over previous
import functools

import jax
import jax.numpy as jnp
from jax import lax
from jax.experimental import pallas as pl
from jax.experimental.pallas import tpu as pltpu
from jax.experimental.pallas import tpu_sc as plsc

_T, _H, _E, _I = 2048, 1024, 8, 2048
_B = 256
_BLOG = 8
_NBP = _T // _B + _E
_PT = _NBP * _B
_IC = 2

_INTERP = False


def _router_body(x_ref, rw_ref, eid_ref, gate_ref):
    x = x_ref[...]
    logits = lax.dot_general(x, rw_ref[...], (((1,), (1,)), ((), ())),
                             preferred_element_type=jnp.float32)
    col = lax.broadcasted_iota(jnp.int32, logits.shape, 1)
    masked = jnp.where(col < _E, logits, -1e30)
    maxv = jnp.max(masked, axis=1)
    eid = jnp.min(jnp.where(masked == maxv[:, None], col, _E), axis=1)
    gate_v = jax.nn.sigmoid(maxv)
    eid_ref[...] = eid.reshape(eid_ref.shape).astype(jnp.int32)
    gate_ref[...] = gate_v.reshape(gate_ref.shape)


def _router(x, rw_pad):
    return pl.pallas_call(
        _router_body,
        out_shape=[
            jax.ShapeDtypeStruct((_T // 128, 128), jnp.int32),
            jax.ShapeDtypeStruct((_T // 128, 128), jnp.float32),
        ],
        interpret=_INTERP,
    )(x, rw_pad)


def _g_body(bexp_ref, xs_ref, pg_ref, w1_ref, w3_ref, w2_ref, ys_ref):
    g = pg_ref[0, 0, :]
    xg = (xs_ref[...] * g[:, None]).astype(jnp.bfloat16)
    w1b = w1_ref[0].astype(jnp.bfloat16)
    w3b = w3_ref[0].astype(jnp.bfloat16)
    h1 = lax.dot_general(xg, w1b, (((1,), (1,)), ((), ())),
                         preferred_element_type=jnp.float32)
    h3 = lax.dot_general(xg, w3b, (((1,), (1,)), ((), ())),
                         preferred_element_type=jnp.float32)
    h = ((h1 * jax.nn.sigmoid(h1)) * h3).astype(jnp.bfloat16)
    ys_ref[...] = lax.dot_general(h, w2_ref[0].astype(jnp.bfloat16),
                                  (((1,), (1,)), ((), ())),
                                  preferred_element_type=jnp.float32)


def _g(bexp, xs, pg2d, w1, w3, w2):
    return pl.pallas_call(
        _g_body,
        grid_spec=pltpu.PrefetchScalarGridSpec(
            num_scalar_prefetch=1,
            grid=(_NBP,),
            in_specs=[
                pl.BlockSpec((_B, _H), lambda b, bexp: (b, 0)),
                pl.BlockSpec((1, 1, _B), lambda b, bexp: (b, 0, 0)),
                pl.BlockSpec((1, _I, _H), lambda b, bexp: (bexp[b], 0, 0)),
                pl.BlockSpec((1, _I, _H), lambda b, bexp: (bexp[b], 0, 0)),
                pl.BlockSpec((1, _H, _I), lambda b, bexp: (bexp[b], 0, 0)),
            ],
            out_specs=pl.BlockSpec((_B, _H), lambda b, bexp: (b, 0)),
        ),
        out_shape=jax.ShapeDtypeStruct((_PT, _H), jnp.float32),
        interpret=_INTERP,
    )(bexp, xs, pg2d, w1, w3, w2)


def _g1_body_unused(bexp_ref, xs_ref, pg_ref, w1_ref, w3_ref, hs_ref):
    pass


def _g1(bexp, xs, pg2d, w1, w3):
    icw = _I // _IC
    return pl.pallas_call(
        _g1_body,
        grid_spec=pltpu.PrefetchScalarGridSpec(
            num_scalar_prefetch=1,
            grid=(_IC, _NBP),
            in_specs=[
                pl.BlockSpec((_B, _H), lambda ic, b, bexp: (b, 0)),
                pl.BlockSpec((1, 1, _B), lambda ic, b, bexp: (b, 0, 0)),
                pl.BlockSpec((1, icw, _H), lambda ic, b, bexp: (bexp[b], ic, 0)),
                pl.BlockSpec((1, icw, _H), lambda ic, b, bexp: (bexp[b], ic, 0)),
            ],
            out_specs=pl.BlockSpec((_B, icw), lambda ic, b, bexp: (b, ic)),
        ),
        out_shape=jax.ShapeDtypeStruct((_PT, _I), jnp.float32),
        interpret=_INTERP,
    )(bexp, xs, pg2d, w1, w3)


def _g2_body(bexp_ref, hs_ref, w2_ref, ys_ref):
    ys_ref[...] = lax.dot_general(hs_ref[...].astype(jnp.bfloat16),
                                  w2_ref[0].astype(jnp.bfloat16),
                                  (((1,), (1,)), ((), ())),
                                  preferred_element_type=jnp.float32)


def _g2(bexp, hs, w2):
    return pl.pallas_call(
        _g2_body,
        grid_spec=pltpu.PrefetchScalarGridSpec(
            num_scalar_prefetch=1,
            grid=(_NBP,),
            in_specs=[
                pl.BlockSpec((_B, _I), lambda b, bexp: (b, 0)),
                pl.BlockSpec((1, _H, _I), lambda b, bexp: (bexp[b], 0, 0)),
            ],
            out_specs=pl.BlockSpec((_B, _H), lambda b, bexp: (b, 0)),
        ),
        out_shape=jax.ShapeDtypeStruct((_PT, _H), jnp.float32),
        interpret=_INTERP,
    )(bexp, hs, w2)


def _s_body(x_ref, sw1_ref, sw3_ref, sw2_ref, y_ref):
    x = x_ref[...].astype(jnp.bfloat16)
    h1 = lax.dot_general(x, sw1_ref[...].astype(jnp.bfloat16),
                         (((1,), (1,)), ((), ())),
                         preferred_element_type=jnp.float32)
    h3 = lax.dot_general(x, sw3_ref[...].astype(jnp.bfloat16),
                         (((1,), (1,)), ((), ())),
                         preferred_element_type=jnp.float32)
    h = ((h1 * jax.nn.sigmoid(h1)) * h3).astype(jnp.bfloat16)
    y_ref[...] = lax.dot_general(h, sw2_ref[...].astype(jnp.bfloat16),
                                 (((1,), (1,)), ((), ())),
                                 preferred_element_type=jnp.float32)


def _s(x, sw1, sw3, sw2):
    tb = 256
    return pl.pallas_call(
        _s_body,
        grid=(_T // tb,),
        in_specs=[
            pl.BlockSpec((tb, _H), lambda b: (b, 0)),
            pl.BlockSpec((_I, _H), lambda b: (0, 0)),
            pl.BlockSpec((_I, _H), lambda b: (0, 0)),
            pl.BlockSpec((_H, _I), lambda b: (0, 0)),
        ],
        out_specs=pl.BlockSpec((tb, _H), lambda b: (b, 0)),
        out_shape=jax.ShapeDtypeStruct((_T, _H), jnp.float32),
        interpret=_INTERP,
    )(x, sw1, sw3, sw2)


def _s1(x, sw1, sw3):
    tb = 256
    icw = _I // _IC
    return pl.pallas_call(
        _s1_body,
        grid=(_IC, _T // tb),
        in_specs=[
            pl.BlockSpec((tb, _H), lambda ic, b: (b, 0)),
            pl.BlockSpec((icw, _H), lambda ic, b: (ic, 0)),
            pl.BlockSpec((icw, _H), lambda ic, b: (ic, 0)),
        ],
        out_specs=pl.BlockSpec((tb, icw), lambda ic, b: (b, ic)),
        out_shape=jax.ShapeDtypeStruct((_T, _I), jnp.float32),
        interpret=_INTERP,
    )(x, sw1, sw3)


def _s2_body(h_ref, sw2_ref, y_ref):
    y_ref[...] = lax.dot_general(
        h_ref[...].astype(jnp.bfloat16), sw2_ref[...].astype(jnp.bfloat16),
        (((1,), (1,)), ((), ())),
        preferred_element_type=jnp.float32)


def _s2(hsh, sw2):
    tb = 256
    return pl.pallas_call(
        _s2_body,
        grid=(_T // tb,),
        in_specs=[
            pl.BlockSpec((tb, _I), lambda b: (b, 0)),
            pl.BlockSpec((_H, _I), lambda b: (0, 0)),
        ],
        out_specs=pl.BlockSpec((tb, _H), lambda b: (b, 0)),
        out_shape=jax.ShapeDtypeStruct((_T, _H), jnp.float32),
        interpret=_INTERP,
    )(hsh, sw2)


def _sc_mesh():
    return plsc.VectorSubcoreMesh(core_axis_name="c", subcore_axis_name="s")


_NW = 32
_L = 16


def _take16(vec, idx):
    dn = lax.GatherDimensionNumbers(offset_dims=(), collapsed_slice_dims=(0,),
                                    start_index_map=(0,))
    return lax.gather(vec, idx[:, None], dn, slice_sizes=(1,),
                      mode=lax.GatherScatterMode.PROMISE_IN_BOUNDS)


def _sc_meta_body(eid_hbm, gate_hbm, pos_hbm, perm_hbm, pgate_hbm, bexp_hbm,
                  eid_v, gate_v, pos_v, perm_v, pgate_v, bexp_v, sem):
    wid = lax.axis_index("s") * 2 + lax.axis_index("c")

    @pl.when(wid == 0)
    def _():
        pltpu.sync_copy(eid_hbm, eid_v)
        pltpu.sync_copy(gate_hbm, gate_v)
        lane = lax.broadcasted_iota(jnp.int32, (_L,), 0)

        def count_step(i, c):
            ech = eid_v[pl.ds(i * _L, _L)]
            for e in range(_E):
                cnt = jnp.sum((ech == e).astype(jnp.int32))
                c = c + jnp.where(lane == e, cnt, 0)
            return c

        counts = lax.fori_loop(0, _T // _L, count_step,
                               jnp.zeros((_L,), jnp.int32))
        aligned = ((counts + (_B - 1)) >> _BLOG) << _BLOG
        incl = plsc.cumsum(aligned)
        excl = incl - aligned
        startblk = excl >> _BLOG

        for k in range(2):
            biota = lane + k * _L
            acc = jnp.zeros((_L,), jnp.int32)
            for e in range(_E):
                sb = _take16(startblk, jnp.full((_L,), e, jnp.int32))
                acc = acc + jnp.where(biota >= sb, 1, 0)
            bexp_v[pl.ds(k * _L, _L)] = jnp.maximum(acc - 1, 0)

        def zero_step(j, _):
            perm_v[pl.ds(j * _L, _L)] = jnp.zeros((_L,), jnp.int32)
            pgate_v[pl.ds(j * _L, _L)] = jnp.zeros((_L,), jnp.float32)
            return 0

        lax.fori_loop(0, _PT // _L, zero_step, 0)

        def pos_step(i, c2):
            ech = eid_v[pl.ds(i * _L, _L)]
            gch = gate_v[pl.ds(i * _L, _L)]
            base = _take16(excl + c2, ech)
            within = jnp.zeros((_L,), jnp.int32)
            cadd = jnp.zeros((_L,), jnp.int32)
            for e in range(_E):
                m = ech == e
                mi = m.astype(jnp.int32)
                cs = plsc.cumsum(mi)
                within = within + mi * cs
                cadd = cadd + jnp.where(lane == e, jnp.sum(mi), 0)
            posch = base + within - 1
            pos_v[pl.ds(i * _L, _L)] = posch
            plsc.store_scatter(perm_v, [posch], lane + i * _L)
            plsc.store_scatter(pgate_v, [posch], gch)
            return c2 + cadd

        lax.fori_loop(0, _T // _L, pos_step, jnp.zeros((_L,), jnp.int32))

        pltpu.sync_copy(pos_v, pos_hbm)
        pltpu.sync_copy(perm_v, perm_hbm)
        pltpu.sync_copy(pgate_v, pgate_hbm)
        pltpu.sync_copy(bexp_v, bexp_hbm)


def _sc_meta(eid, gate):
    f = pl.kernel(
        _sc_meta_body,
        out_type=[
            jax.ShapeDtypeStruct((_T,), jnp.int32),
            jax.ShapeDtypeStruct((_PT,), jnp.int32),
            jax.ShapeDtypeStruct((_PT,), jnp.float32),
            jax.ShapeDtypeStruct((32,), jnp.int32),
        ],
        mesh=_sc_mesh(),
        compiler_params=pltpu.CompilerParams(needs_layout_passes=False),
        scratch_types=[
            pltpu.VMEM((_T,), jnp.int32),
            pltpu.VMEM((_T,), jnp.float32),
            pltpu.VMEM((_T,), jnp.int32),
            pltpu.VMEM((_PT,), jnp.int32),
            pltpu.VMEM((_PT,), jnp.float32),
            pltpu.VMEM((32,), jnp.int32),
            pltpu.SemaphoreType.DMA,
        ],
    )
    return f(eid, gate)


_NBUF = 4
_CH = 16


def _sc_gather_body(src_hbm, idx_hbm, out_hbm, idx_v,
                    b0, b1, b2, b3, g0, g1, g2, g3, w0, w1, w2, w3):
    wid = lax.axis_index("s") * 2 + lax.axis_index("c")
    n = idx_v.shape[0]
    nch = n // _CH
    base = wid * n
    pltpu.sync_copy(idx_hbm.at[pl.ds(base, n)], idx_v)
    bufs, gsems, wsems = [b0, b1, b2, b3], [g0, g1, g2, g3], [w0, w1, w2, w3]
    gd, wd = {}, {}
    for c in range(min(_NBUF, nch)):
        gd[c] = pltpu.async_copy(src_hbm.at[idx_v.at[pl.ds(c * _CH, _CH)]],
                                 bufs[c], gsems[c])
    for c in range(nch):
        b = c % _NBUF
        gd[c].wait()
        wd[c] = pltpu.async_copy(bufs[b],
                                 out_hbm.at[pl.ds(base + c * _CH, _CH)],
                                 wsems[b])
        nxt = c + _NBUF
        if nxt < nch:
            wd[c].wait()
            gd[nxt] = pltpu.async_copy(
                src_hbm.at[idx_v.at[pl.ds(nxt * _CH, _CH)]], bufs[b], gsems[b])
    for c in range(max(0, nch - _NBUF), nch):
        wd[c].wait()


_CHC = 8


def _sc_combine_body(src_hbm, idx_hbm, add_hbm, out_hbm, idx_v,
                     b0, b1, b2, b3, a0, a1, a2, a3,
                     g0, g1, g2, g3, h0, h1, h2, h3, w0, w1, w2, w3):
    wid = lax.axis_index("s") * 2 + lax.axis_index("c")
    n = idx_v.shape[0]
    nch = n // _CHC
    base = wid * n
    pltpu.sync_copy(idx_hbm.at[pl.ds(base, n)], idx_v)
    bufs, abufs = [b0, b1, b2, b3], [a0, a1, a2, a3]
    gsems, asems, wsems = [g0, g1, g2, g3], [h0, h1, h2, h3], [w0, w1, w2, w3]
    gd, ad, wd = {}, {}, {}
    for c in range(min(_NBUF, nch)):
        gd[c] = pltpu.async_copy(src_hbm.at[idx_v.at[pl.ds(c * _CHC, _CHC)]],
                                 bufs[c], gsems[c])
        ad[c] = pltpu.async_copy(add_hbm.at[pl.ds(base + c * _CHC, _CHC)],
                                 abufs[c], asems[c])
    nv = _H // _L
    for c in range(nch):
        b = c % _NBUF
        gd[c].wait()
        ad[c].wait()
        buf, abuf = bufs[b], abufs[b]

        def add_step(r, _, buf=buf, abuf=abuf):
            for k in range(nv):
                sl = pl.ds(k * _L, _L)
                buf[r, sl] = buf[r, sl] + abuf[r, sl]
            return 0

        lax.fori_loop(0, _CHC, add_step, 0)
        wd[c] = pltpu.async_copy(bufs[b],
                                 out_hbm.at[pl.ds(base + c * _CHC, _CHC)],
                                 wsems[b])
        nxt = c + _NBUF
        if nxt < nch:
            wd[c].wait()
            gd[nxt] = pltpu.async_copy(
                src_hbm.at[idx_v.at[pl.ds(nxt * _CHC, _CHC)]], bufs[b], gsems[b])
            ad[nxt] = pltpu.async_copy(
                add_hbm.at[pl.ds(base + nxt * _CHC, _CHC)], abufs[b], asems[b])
    for c in range(max(0, nch - _NBUF), nch):
        wd[c].wait()


def _sc_combine(src, idx, add):
    rows_per = _T // _NW
    f = pl.kernel(
        _sc_combine_body,
        out_type=jax.ShapeDtypeStruct((_T, _H), jnp.float32),
        mesh=_sc_mesh(),
        compiler_params=pltpu.CompilerParams(needs_layout_passes=False),
        scratch_types=(
            [pltpu.VMEM((rows_per,), jnp.int32)]
            + [pltpu.VMEM((_CHC, _H), jnp.float32) for _ in range(2 * _NBUF)]
            + [pltpu.SemaphoreType.DMA for _ in range(3 * _NBUF)]
        ),
    )
    return f(src, idx, add)


def _sc_gather(src, idx, n_out):
    rows_per = n_out // _NW
    f = pl.kernel(
        _sc_gather_body,
        out_type=jax.ShapeDtypeStruct((n_out, _H), jnp.float32),
        mesh=_sc_mesh(),
        compiler_params=pltpu.CompilerParams(needs_layout_passes=False),
        scratch_types=(
            [pltpu.VMEM((rows_per,), jnp.int32)]
            + [pltpu.VMEM((_CH, _H), jnp.float32) for _ in range(_NBUF)]
            + [pltpu.SemaphoreType.DMA for _ in range(2 * _NBUF)]
        ),
    )
    return f(src, idx)


def kernel(hidden_states, router_w, w1, w3, w2, sw1, sw3, sw2):
    x = hidden_states
    eid2d, gate2d = _router(x, router_w)
    eid = eid2d.reshape(_T)
    gate = gate2d.reshape(_T)

    pos, perm, pgate, bexp = _sc_meta(eid, gate)
    pg2d = pgate.reshape(_NBP, 1, _B)

    ysh = _s(x, sw1, sw3, sw2)

    xs = _sc_gather(x, perm, _PT)
    ys = _g(bexp, xs, pg2d, w1, w3, w2)

    return _sc_combine(ys, pos, ysh)
